# baseline (device time: 25930 ns/iter reference)
import jax
import jax.numpy as jnp
from jax import lax
from jax.experimental import pallas as pl
from jax.experimental.pallas import tpu as pltpu

N_DEV = 4
N_HOP = N_DEV - 1


def _gelu(y):
    c = 0.7978845608028654
    return 0.5 * y * (1.0 + jnp.tanh(c * (y + 0.044715 * y * y * y)))


def kernel(x, w_mat):
    m, _ = x.shape
    _, n = w_mat.shape
    m_chunk = m // N_DEV
    n_q = n // 4

    RINGS = (
        dict(lo=0, dirn=+1),
        dict(lo=2 * n_q, dirn=-1),
        dict(lo=1 * n_q, dirn=+1),
        dict(lo=3 * n_q, dirn=-1),
    )

    def body(x_ref, w_ref, out_ref, *scratch):
        ring_refs = scratch[0:4]
        send_sems = scratch[4:8]
        recv_sems = scratch[8:12]

        my = lax.axis_index("i")
        left = (my - 1) % N_DEV
        right = (my + 1) % N_DEV

        barrier = pltpu.get_barrier_semaphore()
        for nbr in (left, right):
            pl.semaphore_signal(
                barrier, inc=1,
                device_id=(nbr,), device_id_type=pl.DeviceIdType.MESH,
            )
        pl.semaphore_wait(barrier, 2)

        def pchunk(c, lo, width):
            return jnp.dot(
                x_ref[pl.ds(c * m_chunk, m_chunk), :],
                w_ref[:, lo:lo + width],
                preferred_element_type=jnp.float32,
            )

        def hop(k, h):
            r = RINGS[k]
            dst = right if r["dirn"] > 0 else left
            return pltpu.make_async_remote_copy(
                src_ref=ring_refs[k].at[h],
                dst_ref=ring_refs[k].at[h + 1],
                send_sem=send_sems[k].at[h],
                recv_sem=recv_sems[k].at[h],
                device_id=(dst,),
                device_id_type=pl.DeviceIdType.MESH,
            )

        rdmas = [[None] * N_HOP for _ in RINGS]
        for k, r in enumerate(RINGS):
            seed = (my - r["dirn"]) % N_DEV
            ring_refs[k][0] = pchunk(seed, r["lo"], n_q)
            rdmas[k][0] = hop(k, 0)
            rdmas[k][0].start()

        ps = [
            pchunk((my + 2) % N_DEV, 0, n),
            jnp.concatenate(
                [
                    pchunk((my + 1) % N_DEV, 0, n // 2),
                    pchunk((my - 1) % N_DEV, n // 2, n // 2),
                ],
                axis=1,
            ),
            pchunk(my, 0, n),
        ]

        for h in range(N_HOP):
            for k, r in enumerate(RINGS):
                rdmas[k][h].wait()
                lo = r["lo"]
                acc = ring_refs[k][h + 1] + ps[h][:, lo:lo + n_q]
                if h + 1 < N_HOP:
                    ring_refs[k][h + 1] = acc
                    rdmas[k][h + 1] = hop(k, h + 1)
                    rdmas[k][h + 1].start()
                else:
                    out_ref[:, lo:lo + n_q] = _gelu(acc)

    return pl.pallas_call(
        body,
        out_shape=jax.ShapeDtypeStruct((m_chunk, n), jnp.float32),
        in_specs=[
            pl.BlockSpec(memory_space=pltpu.VMEM),
            pl.BlockSpec(memory_space=pltpu.VMEM),
        ],
        out_specs=pl.BlockSpec(memory_space=pltpu.VMEM),
        scratch_shapes=(
            [pltpu.VMEM((N_DEV, m_chunk, n_q), jnp.float32)] * 4
            + [pltpu.SemaphoreType.DMA((N_HOP,))] * 8
        ),
        compiler_params=pltpu.CompilerParams(collective_id=0),
    )(x, w_mat)


# device time: 18045 ns/iter; 1.4370x vs baseline; 1.4370x over previous
import jax
import jax.numpy as jnp
from jax import lax
from jax.experimental import pallas as pl
from jax.experimental.pallas import tpu as pltpu

N_DEV = 4
N_HOP = N_DEV - 1


def _gelu(y):
    c = 0.7978845608028654
    return 0.5 * y * (1.0 + jnp.tanh(c * (y + 0.044715 * y * y * y)))


def kernel(x, w_mat):
    m, _ = x.shape
    _, n = w_mat.shape
    m_chunk = m // N_DEV
    n_q = n // 4

    RINGS = (
        dict(lo=0, dirn=+1),
        dict(lo=2 * n_q, dirn=-1),
        dict(lo=1 * n_q, dirn=+1),
        dict(lo=3 * n_q, dirn=-1),
    )

    def body(x_ref, w_ref, out_ref, *scratch):
        ring_refs = scratch[0:4]
        send_sems = scratch[4:8]
        recv_sems = scratch[8:12]

        my = lax.axis_index("i")
        left = (my - 1) % N_DEV
        right = (my + 1) % N_DEV

        barrier = pltpu.get_barrier_semaphore()
        for nbr in (left, right):
            pl.semaphore_signal(
                barrier, inc=1,
                device_id=(nbr,), device_id_type=pl.DeviceIdType.MESH,
            )
        pl.semaphore_wait(barrier, 2)

        def pchunk(c, lo, width):
            return jnp.dot(
                x_ref[pl.ds(c * m_chunk, m_chunk), :],
                w_ref[:, lo:lo + width],
                preferred_element_type=jnp.float32,
            )

        def hop(k, h):
            r = RINGS[k]
            dst = right if r["dirn"] > 0 else left
            return pltpu.make_async_remote_copy(
                src_ref=ring_refs[k].at[h],
                dst_ref=ring_refs[k].at[h + 1],
                send_sem=send_sems[k].at[h],
                recv_sem=recv_sems[k].at[h],
                device_id=(dst,),
                device_id_type=pl.DeviceIdType.MESH,
            )

        rdmas = [[None] * N_HOP for _ in RINGS]
        for k, r in enumerate(RINGS):
            seed = (my - r["dirn"]) % N_DEV
            ring_refs[k][0] = pchunk(seed, r["lo"], n_q).astype(jnp.bfloat16)
            rdmas[k][0] = hop(k, 0)
            rdmas[k][0].start()

        ps = [
            pchunk((my + 2) % N_DEV, 0, n),
            jnp.concatenate(
                [
                    pchunk((my + 1) % N_DEV, 0, n // 2),
                    pchunk((my - 1) % N_DEV, n // 2, n // 2),
                ],
                axis=1,
            ),
            pchunk(my, 0, n),
        ]

        for h in range(N_HOP):
            for k, r in enumerate(RINGS):
                rdmas[k][h].wait_recv()
                lo = r["lo"]
                acc = (
                    ring_refs[k][h + 1].astype(jnp.float32)
                    + ps[h][:, lo:lo + n_q]
                )
                if h + 1 < N_HOP:
                    ring_refs[k][h + 1] = acc.astype(jnp.bfloat16)
                    rdmas[k][h + 1] = hop(k, h + 1)
                    rdmas[k][h + 1].start()
                else:
                    out_ref[:, lo:lo + n_q] = _gelu(acc)

        for h in range(N_HOP):
            for k in range(len(RINGS)):
                rdmas[k][h].wait_send()

    return pl.pallas_call(
        body,
        out_shape=jax.ShapeDtypeStruct((m_chunk, n), jnp.float32),
        in_specs=[
            pl.BlockSpec(memory_space=pltpu.VMEM),
            pl.BlockSpec(memory_space=pltpu.VMEM),
        ],
        out_specs=pl.BlockSpec(memory_space=pltpu.VMEM),
        scratch_shapes=(
            [pltpu.VMEM((N_DEV, m_chunk, n_q), jnp.bfloat16)] * 4
            + [pltpu.SemaphoreType.DMA((N_HOP,))] * 8
        ),
        compiler_params=pltpu.CompilerParams(collective_id=0),
    )(x, w_mat)


# device time: 17786 ns/iter; 1.4579x vs baseline; 1.0146x over previous
import jax
import jax.numpy as jnp
from jax import lax
from jax.experimental import pallas as pl
from jax.experimental.pallas import tpu as pltpu

N_DEV = 4


def _gelu(y):
    c = 0.7978845608028654
    return 0.5 * y * (1.0 + jnp.tanh(c * (y + 0.044715 * y * y * y)))


def kernel(x, w_mat):
    m, _ = x.shape
    _, n = w_mat.shape
    mc = m // N_DEV
    nh = n // 2

    def body(x_ref, w_ref, out_ref,
             sbufA, sbufB, rbufA, rbufB, s2A, s2B, r2A, r2B,
             ssemA, ssemB, rsemA, rsemB,
             ssem2A, ssem2B, rsem2A, rsem2B):
        my = lax.axis_index("i")
        pA = jnp.bitwise_xor(my, 1)
        pB = 3 - my

        barrier = pltpu.get_barrier_semaphore()
        for nbr in (pA, pB):
            pl.semaphore_signal(
                barrier, inc=1,
                device_id=(nbr,), device_id_type=pl.DeviceIdType.MESH,
            )
        pl.semaphore_wait(barrier, 2)

        def pchunk(c, lo, width):
            return jnp.dot(
                x_ref[pl.ds(c * mc, mc), :],
                w_ref[:, lo:lo + width],
                preferred_element_type=jnp.float32,
            )

        def xfer(src, dst, ssem, rsem, peer):
            return pltpu.make_async_remote_copy(
                src_ref=src, dst_ref=dst, send_sem=ssem, recv_sem=rsem,
                device_id=(peer,), device_id_type=pl.DeviceIdType.MESH,
            )

        sbufA[0] = pchunk(3 - pA, 0, nh).astype(jnp.bfloat16)
        ra0 = xfer(sbufA.at[0], rbufA.at[0], ssemA.at[0], rsemA.at[0], pA)
        ra0.start()
        sbufB[0] = pchunk(jnp.bitwise_xor(pB, 1), nh, nh).astype(jnp.bfloat16)
        rb0 = xfer(sbufB.at[0], rbufB.at[0], ssemB.at[0], rsemB.at[0], pB)
        rb0.start()
        sbufA[1] = pchunk(pA, 0, nh).astype(jnp.bfloat16)
        ra1 = xfer(sbufA.at[1], rbufA.at[1], ssemA.at[1], rsemA.at[1], pA)
        ra1.start()
        sbufB[1] = pchunk(pB, nh, nh).astype(jnp.bfloat16)
        rb1 = xfer(sbufB.at[1], rbufB.at[1], ssemB.at[1], rsemB.at[1], pB)
        rb1.start()

        p_fwd_a = pchunk(3 - my, 0, nh)
        p_fwd_b = pchunk(pA, nh, nh)

        ra0.wait_recv()
        s2A[...] = (rbufA[0].astype(jnp.float32) + p_fwd_a).astype(jnp.bfloat16)
        r2a = xfer(s2A, r2A, ssem2A, rsem2A, pB)
        r2a.start()
        rb0.wait_recv()
        s2B[...] = (rbufB[0].astype(jnp.float32) + p_fwd_b).astype(jnp.bfloat16)
        r2b = xfer(s2B, r2B, ssem2B, rsem2B, pA)
        r2b.start()

        p_own = pchunk(my, 0, n)

        ra1.wait_recv()
        kept_a = rbufA[1].astype(jnp.float32) + p_own[:, :nh]
        rb1.wait_recv()
        kept_b = rbufB[1].astype(jnp.float32) + p_own[:, nh:]

        r2a.wait_recv()
        out_ref[:, :nh] = _gelu(kept_a + r2A[...].astype(jnp.float32))
        r2b.wait_recv()
        out_ref[:, nh:] = _gelu(kept_b + r2B[...].astype(jnp.float32))

        for r in (ra0, rb0, ra1, rb1, r2a, r2b):
            r.wait_send()

    return pl.pallas_call(
        body,
        out_shape=jax.ShapeDtypeStruct((mc, n), jnp.float32),
        in_specs=[
            pl.BlockSpec(memory_space=pltpu.VMEM),
            pl.BlockSpec(memory_space=pltpu.VMEM),
        ],
        out_specs=pl.BlockSpec(memory_space=pltpu.VMEM),
        scratch_shapes=(
            [pltpu.VMEM((2, mc, nh), jnp.bfloat16)] * 4
            + [pltpu.VMEM((mc, nh), jnp.bfloat16)] * 4
            + [pltpu.SemaphoreType.DMA((2,))] * 4
            + [pltpu.SemaphoreType.DMA] * 4
        ),
        compiler_params=pltpu.CompilerParams(collective_id=0),
    )(x, w_mat)
